# baseline (speedup 1.0000x reference)
import jax
import jax.numpy as jnp
from jax.experimental import pallas as pl
from jax.experimental.pallas import tpu as pltpu

T = 4
BLOCKS = 8
D = 1024
N_TOK = 4096
TN = 256
CH = 8


def _gating_kernel(g_ref, x_ref, w_ref, o_ref, h_ref):
    g = g_ref[:]
    m = jnp.max(g, axis=-1, keepdims=True)
    e = jnp.exp(g - m)
    probs = e / jnp.sum(e, axis=-1, keepdims=True)

    x = x_ref[:].astype(jnp.bfloat16)
    for b in range(BLOCKS):
        h_ref[b] = jnp.maximum(
            jnp.dot(
                x,
                w_ref[b].astype(jnp.bfloat16),
                preferred_element_type=jnp.float32,
            ),
            0.0,
        )

    def chunk_body(i, carry):
        r = i * CH
        hc = [h_ref[b, pl.ds(r, CH), :] for b in range(BLOCKS)]
        for t in range(T):
            acc = probs[t : t + 1, 0:1] * hc[0]
            for b in range(1, BLOCKS):
                acc = acc + probs[t : t + 1, b : b + 1] * hc[b]
            o_ref[t, pl.ds(r, CH), :] = acc
        return carry

    jax.lax.fori_loop(0, TN // CH, chunk_body, 0)


def kernel(inputs, W, g_logits):
    g_pad = jnp.full((8, 128), -1e30, dtype=jnp.float32)
    g_pad = jax.lax.dynamic_update_slice(g_pad, g_logits, (0, 0))

    grid = (N_TOK // TN,)
    out = pl.pallas_call(
        _gating_kernel,
        grid=grid,
        in_specs=[
            pl.BlockSpec((8, 128), lambda n: (0, 0)),
            pl.BlockSpec((TN, D), lambda n: (n, 0)),
            pl.BlockSpec((BLOCKS, D, D), lambda n: (0, 0, 0)),
        ],
        out_specs=pl.BlockSpec((T, TN, D), lambda n: (0, n, 0)),
        out_shape=jax.ShapeDtypeStruct((T, N_TOK, D), jnp.float32),
        scratch_shapes=[pltpu.VMEM((BLOCKS, TN, D), jnp.float32)],
    )(g_pad, inputs, W)
    return out


# speedup vs baseline: 1.4143x; 1.4143x over previous
import jax
import jax.numpy as jnp
from jax.experimental import pallas as pl
from jax.experimental.pallas import tpu as pltpu

T = 4
BLOCKS = 8
D = 1024
N_TOK = 4096
TN = 512


def _gating_kernel(g_ref, x_ref, w_hbm, o_ref, w_vmem, sems):
    g = g_ref[:]
    m = jnp.max(g, axis=-1, keepdims=True)
    e = jnp.exp(g - m)
    probs = e / jnp.sum(e, axis=-1, keepdims=True)

    n = pl.program_id(0)

    @pl.when(n == 0)
    def _():
        for b in range(BLOCKS):
            pltpu.make_async_copy(
                w_hbm.at[b], w_vmem.at[b], sems.at[b]
            ).start()

    x = x_ref[:].astype(jnp.bfloat16)
    for b in range(BLOCKS):

        @pl.when(n == 0)
        def _():
            pltpu.make_async_copy(
                w_hbm.at[b], w_vmem.at[b], sems.at[b]
            ).wait()

        h = jnp.maximum(
            jnp.dot(
                x,
                w_vmem[b].astype(jnp.bfloat16),
                preferred_element_type=jnp.float32,
            ),
            0.0,
        )
        for t in range(T):
            p = probs[t : t + 1, b : b + 1]
            if b == 0:
                o_ref[t] = p * h
            else:
                o_ref[t] += p * h


def kernel(inputs, W, g_logits):
    g_pad = jnp.full((8, 128), -1e30, dtype=jnp.float32)
    g_pad = jax.lax.dynamic_update_slice(g_pad, g_logits, (0, 0))

    grid = (N_TOK // TN,)
    out = pl.pallas_call(
        _gating_kernel,
        grid=grid,
        in_specs=[
            pl.BlockSpec((8, 128), lambda n: (0, 0)),
            pl.BlockSpec((TN, D), lambda n: (n, 0)),
            pl.BlockSpec(memory_space=pl.ANY),
        ],
        out_specs=pl.BlockSpec((T, TN, D), lambda n: (0, n, 0)),
        out_shape=jax.ShapeDtypeStruct((T, N_TOK, D), jnp.float32),
        scratch_shapes=[
            pltpu.VMEM((BLOCKS, D, D), jnp.float32),
            pltpu.SemaphoreType.DMA((BLOCKS,)),
        ],
    )(g_pad, inputs, W)
    return out


# speedup vs baseline: 1.4763x; 1.0438x over previous
import jax
import jax.numpy as jnp
from jax.experimental import pallas as pl
from jax.experimental.pallas import tpu as pltpu

T = 4
BLOCKS = 8
D = 1024
N_TOK = 4096
TN = 512


def _gating_kernel(g_ref, x_ref, w_ref, o_ref):
    g = g_ref[:]
    m = jnp.max(g, axis=-1, keepdims=True)
    e = jnp.exp(g - m)
    probs = e / jnp.sum(e, axis=-1, keepdims=True)

    x = x_ref[:].astype(jnp.bfloat16)
    for b in range(BLOCKS):
        h = jnp.maximum(
            jnp.dot(
                x,
                w_ref[b].astype(jnp.bfloat16),
                preferred_element_type=jnp.float32,
            ),
            0.0,
        )
        for t in range(T):
            p = probs[t : t + 1, b : b + 1]
            if b == 0:
                o_ref[t] = p * h
            else:
                o_ref[t] += p * h


def kernel(inputs, W, g_logits):
    grid = (N_TOK // TN,)
    out = pl.pallas_call(
        _gating_kernel,
        grid=grid,
        in_specs=[
            pl.BlockSpec((T, BLOCKS), lambda n: (0, 0)),
            pl.BlockSpec((TN, D), lambda n: (n, 0)),
            pl.BlockSpec((BLOCKS, D, D), lambda n: (0, 0, 0)),
        ],
        out_specs=pl.BlockSpec((T, TN, D), lambda n: (0, n, 0)),
        out_shape=jax.ShapeDtypeStruct((T, N_TOK, D), jnp.float32),
    )(g_logits, inputs, W)
    return out


# speedup vs baseline: 1.5837x; 1.0728x over previous
import jax
import jax.numpy as jnp
from jax.experimental import pallas as pl
from jax.experimental.pallas import tpu as pltpu

T = 4
BLOCKS = 8
D = 1024
N_TOK = 4096
TN = 1024
DH = D // 2


def _gating_kernel(g_ref, x_ref, w_ref, o_ref):
    g = g_ref[:]
    m = jnp.max(g, axis=-1, keepdims=True)
    e = jnp.exp(g - m)
    probs = e / jnp.sum(e, axis=-1, keepdims=True)

    x = x_ref[:].astype(jnp.bfloat16)
    for b in range(BLOCKS):
        h = jnp.maximum(
            jnp.dot(
                x,
                w_ref[b].astype(jnp.bfloat16),
                preferred_element_type=jnp.float32,
            ),
            0.0,
        )
        for t in range(T):
            p = probs[t : t + 1, b : b + 1]
            if b == 0:
                o_ref[t] = p * h
            else:
                o_ref[t] += p * h


def kernel(inputs, W, g_logits):
    grid = (2, N_TOK // TN)
    out = pl.pallas_call(
        _gating_kernel,
        grid=grid,
        in_specs=[
            pl.BlockSpec((T, BLOCKS), lambda j, n: (0, 0)),
            pl.BlockSpec((TN, D), lambda j, n: (n, 0)),
            pl.BlockSpec((BLOCKS, D, DH), lambda j, n: (0, 0, j)),
        ],
        out_specs=pl.BlockSpec((T, TN, DH), lambda j, n: (0, n, j)),
        out_shape=jax.ShapeDtypeStruct((T, N_TOK, D), jnp.float32),
    )(g_logits, inputs, W)
    return out
